# baseline (speedup 1.0000x reference)
import functools

import jax
import jax.numpy as jnp
from jax.experimental import pallas as pl
from jax.experimental.pallas import tpu as pltpu

_PREC = jax.lax.Precision.DEFAULT


def _gnn_body(adj_ref, w_ref, f_ref, out_ref, p_ref):
    @pl.when(pl.program_id(1) == 0)
    def _():
        p_ref[...] = jnp.dot(adj_ref[...], w_ref[...],
                             preferred_element_type=jnp.float32,
                             precision=_PREC)

    o = jnp.dot(p_ref[...], f_ref[...],
                preferred_element_type=jnp.float32, precision=_PREC)
    out_ref[...] = jnp.maximum(o, 0.0)


@functools.partial(jax.jit, static_argnames=("block_m", "block_n"))
def _gnn(features, adj, weight, block_m=512, block_n=2048):
    n, in_f = adj.shape
    out_f = features.shape[0]
    n_out = features.shape[1]
    grid = (n // block_m, n_out // block_n)
    return pl.pallas_call(
        _gnn_body,
        grid=grid,
        in_specs=[
            pl.BlockSpec((block_m, in_f), lambda i, j: (i, 0)),
            pl.BlockSpec((in_f, out_f), lambda i, j: (0, 0)),
            pl.BlockSpec((out_f, block_n), lambda i, j: (0, j)),
        ],
        out_specs=pl.BlockSpec((block_m, block_n), lambda i, j: (i, j)),
        out_shape=jax.ShapeDtypeStruct((n, n_out), jnp.float32),
        scratch_shapes=[pltpu.VMEM((block_m, out_f), jnp.float32)],
        compiler_params=pltpu.CompilerParams(
            dimension_semantics=("arbitrary", "arbitrary"),
        ),
    )(adj, weight, features)


def kernel(features, adj, weight):
    return _gnn(features, adj, weight)


# speedup vs baseline: 1.5296x; 1.5296x over previous
import functools

import jax
import jax.numpy as jnp
from jax.experimental import pallas as pl
from jax.experimental.pallas import tpu as pltpu

_PREC = jax.lax.Precision.DEFAULT

_CHUNKS = [(0, 256)] + [(256 + 512 * k, 512) for k in range(7)] + [(3840, 256)]
_NA = 3
_NO = 2


def _gnn_body(adj_hbm, w_ref, f_ref, out_hbm, a_buf, o_buf, in_sems, out_sems):
    def in_copy(ci):
        off, sz = _CHUNKS[ci]
        return pltpu.make_async_copy(
            adj_hbm.at[pl.ds(off, sz)],
            a_buf.at[ci % _NA, pl.ds(0, sz)],
            in_sems.at[ci % _NA])

    def out_copy(ci):
        off, sz = _CHUNKS[ci]
        return pltpu.make_async_copy(
            o_buf.at[ci % _NO, pl.ds(0, sz)],
            out_hbm.at[pl.ds(off, sz)],
            out_sems.at[ci % _NO])

    n_c = len(_CHUNKS)
    in_copy(0).start()
    in_copy(1).start()
    for i in range(n_c):
        if i + 2 < n_c:
            in_copy(i + 2).start()
        in_copy(i).wait()
        if i >= _NO:
            out_copy(i - _NO).wait()
        _, sz = _CHUNKS[i]
        a = a_buf[i % _NA, 0:sz]
        p = jnp.dot(a, w_ref[...],
                    preferred_element_type=jnp.float32, precision=_PREC)
        o_buf[i % _NO, 0:sz] = jnp.maximum(
            jnp.dot(p, f_ref[...],
                    preferred_element_type=jnp.float32, precision=_PREC),
            0.0)
        out_copy(i).start()
    out_copy(n_c - 2).wait()
    out_copy(n_c - 1).wait()


@jax.jit
def _gnn(features, adj, weight):
    n, in_f = adj.shape
    out_f = features.shape[0]
    n_out = features.shape[1]
    max_sz = max(sz for _, sz in _CHUNKS)
    return pl.pallas_call(
        _gnn_body,
        in_specs=[
            pl.BlockSpec(memory_space=pltpu.MemorySpace.HBM),
            pl.BlockSpec(memory_space=pltpu.MemorySpace.VMEM),
            pl.BlockSpec(memory_space=pltpu.MemorySpace.VMEM),
        ],
        out_specs=pl.BlockSpec(memory_space=pltpu.MemorySpace.HBM),
        out_shape=jax.ShapeDtypeStruct((n, n_out), jnp.float32),
        scratch_shapes=[
            pltpu.VMEM((_NA, max_sz, in_f), jnp.float32),
            pltpu.VMEM((_NO, max_sz, n_out), jnp.float32),
            pltpu.SemaphoreType.DMA((_NA,)),
            pltpu.SemaphoreType.DMA((_NO,)),
        ],
    )(adj, weight, features)


def kernel(features, adj, weight):
    return _gnn(features, adj, weight)
